# baseline (device time: 16079 ns/iter reference)
import jax
import jax.numpy as jnp
from jax import lax
from jax.experimental import pallas as pl
from jax.experimental.pallas import tpu as pltpu

N_DEV = 16
BLK = 128
CHUNK = 256


def kernel(x):
    m, n = x.shape
    nblk = m // BLK
    nchunk = m // CHUNK

    def body(
        x_hbm, out_ref, xv, send_buf, recv_bufs,
        in_sems, send_sems, recv_sems,
    ):
        my_i = lax.axis_index("i")

        barrier_sem = pltpu.get_barrier_semaphore()
        for k in range(1, N_DEV):
            pl.semaphore_signal(
                barrier_sem, inc=1,
                device_id=(lax.rem(my_i + k, N_DEV),),
                device_id_type=pl.DeviceIdType.MESH,
            )

        in_copies = [
            pltpu.make_async_copy(
                x_hbm.at[pl.ds(c * CHUNK, CHUNK), :],
                xv.at[pl.ds(c * CHUNK, CHUNK), :],
                in_sems.at[c],
            )
            for c in range(nchunk)
        ]
        for cp in in_copies:
            cp.start()
        total = jnp.zeros((1, n), jnp.float32)
        for c, cp in enumerate(in_copies):
            cp.wait()
            total = total + jnp.sum(
                xv[pl.ds(c * CHUNK, CHUNK), :], axis=0, keepdims=True
            )
        send_buf[:, :] = total

        pl.semaphore_wait(barrier_sem, N_DEV - 1)
        for k in range(1, N_DEV):
            pltpu.make_async_remote_copy(
                src_ref=send_buf,
                dst_ref=recv_bufs.at[k],
                send_sem=send_sems.at[k],
                recv_sem=recv_sems.at[k],
                device_id=(lax.rem(my_i + k, N_DEV),),
                device_id_type=pl.DeviceIdType.MESH,
            ).start()

        row = lax.broadcasted_iota(jnp.int32, (BLK, BLK), 0)
        col = lax.broadcasted_iota(jnp.int32, (BLK, BLK), 1)
        ltri = jnp.where(row >= col, 1.0, 0.0).astype(jnp.bfloat16)
        carries = []
        acc = jnp.zeros((1, n), jnp.float32)
        for b in range(nblk):
            blk = xv[pl.ds(b * BLK, BLK), :].astype(jnp.bfloat16)
            d = jnp.dot(ltri, blk, preferred_element_type=jnp.float32)
            out_ref[pl.ds(b * BLK, BLK), :] = d
            carries.append(acc)
            acc = acc + d[BLK - 1:BLK, :]

        for k in range(1, N_DEV):
            pltpu.make_async_remote_copy(
                src_ref=send_buf,
                dst_ref=recv_bufs.at[k],
                send_sem=send_sems.at[k],
                recv_sem=recv_sems.at[k],
                device_id=(lax.rem(my_i - k + N_DEV, N_DEV),),
                device_id_type=pl.DeviceIdType.MESH,
            ).wait_recv()
        for k in range(1, N_DEV):
            pltpu.make_async_remote_copy(
                src_ref=send_buf,
                dst_ref=recv_bufs.at[k],
                send_sem=send_sems.at[k],
                recv_sem=recv_sems.at[k],
                device_id=(lax.rem(my_i + k, N_DEV),),
                device_id_type=pl.DeviceIdType.MESH,
            ).wait_send()

        slots = recv_bufs[:, 0, :]
        kidx = lax.broadcasted_iota(jnp.int32, (N_DEV, n), 0)
        masked = jnp.where((kidx >= 1) & (kidx <= my_i), slots, 0.0)
        excl = jnp.sum(masked, axis=0, keepdims=True)

        for b in range(nblk):
            out_ref[pl.ds(b * BLK, BLK), :] = (
                out_ref[pl.ds(b * BLK, BLK), :] + (excl + carries[b])
            )

    return pl.pallas_call(
        body,
        out_shape=jax.ShapeDtypeStruct((m, n), jnp.float32),
        in_specs=[pl.BlockSpec(memory_space=pl.ANY)],
        out_specs=pl.BlockSpec(memory_space=pltpu.MemorySpace.VMEM),
        scratch_shapes=[
            pltpu.VMEM((m, n), jnp.float32),
            pltpu.VMEM((1, n), jnp.float32),
            pltpu.VMEM((N_DEV, 1, n), jnp.float32),
            pltpu.SemaphoreType.DMA((m // CHUNK,)),
            pltpu.SemaphoreType.DMA((N_DEV,)),
            pltpu.SemaphoreType.DMA((N_DEV,)),
        ],
        compiler_params=pltpu.CompilerParams(collective_id=0),
    )(x)
